# Initial kernel scaffold; baseline (speedup 1.0000x reference)
import jax
import jax.numpy as jnp
from jax.experimental import pallas as pl


def kernel(phase, amplitude, temperature):
    raise NotImplementedError("write your pallas kernel here")


# speedup vs baseline: 7.8001x; 7.8001x over previous
import functools
import math

import jax
import jax.numpy as jnp
from jax.experimental import pallas as pl

_N_OSC = 32768
_K = max(1, int(0.1 * _N_OSC))
_ROWS_PER_BLOCK = 8
_N_ROWS = 128
_HI0 = 0x3F800000


def _body(phase_ref, amp_ref, out_ref):
    phase = phase_ref[...]
    amp = amp_ref[...]
    act = amp * (0.5 * (1.0 + jnp.cos(phase)))
    bits = jax.lax.bitcast_convert_type(act, jnp.int32)

    r = act.shape[0]
    lo0 = jnp.zeros((r, 1), jnp.int32)
    hi0 = jnp.full((r, 1), _HI0, jnp.int32)

    def step(_, carry):
        lo, hi = carry
        mid = jax.lax.div(lo + hi, 2)
        cnt = jnp.sum((bits >= mid).astype(jnp.int32), axis=-1, keepdims=True)
        pred = cnt >= _K
        lo = jnp.where(pred, mid, lo)
        hi = jnp.where(pred, hi, mid)
        return lo, hi

    lo, _ = jax.lax.fori_loop(0, 30, step, (lo0, hi0))
    out_ref[...] = jnp.where(bits >= lo, act, 0.0)


def kernel(phase, amplitude, temperature):
    del temperature
    grid = (_N_ROWS // _ROWS_PER_BLOCK,)
    spec = pl.BlockSpec((_ROWS_PER_BLOCK, _N_OSC), lambda i: (i, 0))
    return pl.pallas_call(
        _body,
        grid=grid,
        in_specs=[spec, spec],
        out_specs=spec,
        out_shape=jax.ShapeDtypeStruct((_N_ROWS, _N_OSC), jnp.float32),
    )(phase, amplitude)
